# baseline (speedup 1.0000x reference)
import jax
import jax.numpy as jnp
from jax.experimental import pallas as pl
from jax.experimental.pallas import tpu as pltpu

_SPLIT = 5


def _gc_kernel(v_ref, w_ref, b_ref, *refs):
    ads = refs[:_SPLIT]
    aas = refs[_SPLIT:2 * _SPLIT]
    o_ref = refs[2 * _SPLIT]
    s_ref = refs[2 * _SPLIT + 1]

    @pl.when(pl.program_id(0) == 0)
    def _():
        s_ref[...] = jnp.dot(v_ref[...].astype(jnp.bfloat16),
                             w_ref[...].astype(jnp.bfloat16),
                             preferred_element_type=jnp.float32
                             ).astype(jnp.bfloat16)

    sub = ads[0].shape[0]
    for j in range(_SPLIT):
        a = (ads[j][...] + aas[j][...]).astype(jnp.bfloat16)
        o_ref[j * sub:(j + 1) * sub, :] = (
            jnp.dot(a, s_ref[...], preferred_element_type=jnp.float32)
            + b_ref[...])


def kernel(vertex, adj_distance, adj_angle, weights, bias):
    n, in_f = vertex.shape
    out_f = weights.shape[1]
    bias2 = bias.reshape(1, out_f)

    tm = 200
    sub = tm // _SPLIT
    grid = (n // tm,)

    def _sub_spec(j):
        return pl.BlockSpec((sub, n), lambda m, j=j: (_SPLIT * m + j, 0))

    return pl.pallas_call(
        _gc_kernel,
        grid=grid,
        in_specs=(
            [pl.BlockSpec((n, in_f), lambda m: (0, 0)),
             pl.BlockSpec((in_f, out_f), lambda m: (0, 0)),
             pl.BlockSpec((1, out_f), lambda m: (0, 0))]
            + [_sub_spec(j) for j in range(_SPLIT)]
            + [_sub_spec(j) for j in range(_SPLIT)]
        ),
        out_specs=pl.BlockSpec((tm, out_f), lambda m: (m, 0)),
        out_shape=jax.ShapeDtypeStruct((n, out_f), jnp.float32),
        scratch_shapes=[pltpu.VMEM((n, out_f), jnp.bfloat16)],
        compiler_params=pltpu.CompilerParams(
            dimension_semantics=("arbitrary",),
        ),
    )(vertex, weights, bias2,
      *([adj_distance] * _SPLIT), *([adj_angle] * _SPLIT))


# speedup vs baseline: 1.0923x; 1.0923x over previous
import jax
import jax.numpy as jnp
from jax.experimental import pallas as pl
from jax.experimental.pallas import tpu as pltpu


def _gc_kernel(v_ref, w_ref, b_ref, ad_ref, aa_ref, o_ref, s_ref):
    @pl.when(pl.program_id(0) == 0)
    def _():
        s_ref[...] = jnp.dot(v_ref[...].astype(jnp.bfloat16),
                             w_ref[...].astype(jnp.bfloat16),
                             preferred_element_type=jnp.float32
                             ).astype(jnp.bfloat16)

    a = (ad_ref[...] + aa_ref[...]).astype(jnp.bfloat16)
    o_ref[...] = (jnp.dot(a, s_ref[...], preferred_element_type=jnp.float32)
                  + b_ref[...])


def kernel(vertex, adj_distance, adj_angle, weights, bias):
    n, in_f = vertex.shape
    out_f = weights.shape[1]
    bias2 = bias.reshape(1, out_f)

    tm = 200
    grid = (n // tm,)

    return pl.pallas_call(
        _gc_kernel,
        grid=grid,
        in_specs=[
            pl.BlockSpec((n, in_f), lambda m: (0, 0)),
            pl.BlockSpec((in_f, out_f), lambda m: (0, 0)),
            pl.BlockSpec((1, out_f), lambda m: (0, 0)),
            pl.BlockSpec((tm, n), lambda m: (m, 0)),
            pl.BlockSpec((tm, n), lambda m: (m, 0)),
        ],
        out_specs=pl.BlockSpec((tm, out_f), lambda m: (m, 0)),
        out_shape=jax.ShapeDtypeStruct((n, out_f), jnp.float32),
        scratch_shapes=[pltpu.VMEM((n, out_f), jnp.bfloat16)],
        compiler_params=pltpu.CompilerParams(
            dimension_semantics=("arbitrary",),
        ),
    )(vertex, weights, bias2, adj_distance, adj_angle)


# speedup vs baseline: 1.1128x; 1.0187x over previous
import functools

import jax
import jax.numpy as jnp
from jax import lax
from jax.experimental import pallas as pl
from jax.experimental.pallas import tpu as pltpu

_NBUF = 4
_ROWS = 80


def _gc_kernel(v_ref, w_ref, b_ref, ad_hbm, aa_hbm, o_ref,
               s_ref, ad_buf, aa_buf, ad_sem, aa_sem, *, n_chunks):
    def _start(chunk, slot):
        pltpu.make_async_copy(
            ad_hbm.at[pl.ds(chunk * _ROWS, _ROWS), :],
            ad_buf.at[slot], ad_sem.at[slot]).start()
        pltpu.make_async_copy(
            aa_hbm.at[pl.ds(chunk * _ROWS, _ROWS), :],
            aa_buf.at[slot], aa_sem.at[slot]).start()

    for slot in range(_NBUF):
        _start(slot, slot)

    s_ref[...] = jnp.dot(v_ref[...].astype(jnp.bfloat16),
                         w_ref[...].astype(jnp.bfloat16),
                         preferred_element_type=jnp.float32
                         ).astype(jnp.bfloat16)

    def _body(i, carry):
        slot = lax.rem(i, _NBUF)
        pltpu.make_async_copy(
            ad_hbm.at[pl.ds(i * _ROWS, _ROWS), :],
            ad_buf.at[slot], ad_sem.at[slot]).wait()
        pltpu.make_async_copy(
            aa_hbm.at[pl.ds(i * _ROWS, _ROWS), :],
            aa_buf.at[slot], aa_sem.at[slot]).wait()
        a = (ad_buf[slot] + aa_buf[slot]).astype(jnp.bfloat16)
        o_ref[pl.ds(i * _ROWS, _ROWS), :] = (
            jnp.dot(a, s_ref[...], preferred_element_type=jnp.float32)
            + b_ref[...])

        @pl.when(i + _NBUF < n_chunks)
        def _():
            _start(i + _NBUF, slot)

        return carry

    lax.fori_loop(0, n_chunks, _body, 0, unroll=False)


def kernel(vertex, adj_distance, adj_angle, weights, bias):
    n, in_f = vertex.shape
    out_f = weights.shape[1]
    bias2 = bias.reshape(1, out_f)
    n_chunks = n // _ROWS

    return pl.pallas_call(
        functools.partial(_gc_kernel, n_chunks=n_chunks),
        in_specs=[
            pl.BlockSpec(memory_space=pltpu.VMEM),
            pl.BlockSpec(memory_space=pltpu.VMEM),
            pl.BlockSpec(memory_space=pltpu.VMEM),
            pl.BlockSpec(memory_space=pl.ANY),
            pl.BlockSpec(memory_space=pl.ANY),
        ],
        out_specs=pl.BlockSpec(memory_space=pltpu.VMEM),
        out_shape=jax.ShapeDtypeStruct((n, out_f), jnp.float32),
        scratch_shapes=[
            pltpu.VMEM((n, out_f), jnp.bfloat16),
            pltpu.VMEM((_NBUF, _ROWS, n), jnp.float32),
            pltpu.VMEM((_NBUF, _ROWS, n), jnp.float32),
            pltpu.SemaphoreType.DMA((_NBUF,)),
            pltpu.SemaphoreType.DMA((_NBUF,)),
        ],
    )(vertex, weights, bias2, adj_distance, adj_angle)
